# Initial kernel scaffold; baseline (speedup 1.0000x reference)
import jax
import jax.numpy as jnp
from jax.experimental import pallas as pl


def kernel(x, edge_index, batch, W1_0, b1_0, W2_0, b2_0, W1_1, b1_1, W2_1, b2_1, W1_2, b1_2, W2_2, b2_2, W_mu, b_mu, W_lv, b_lv):
    raise NotImplementedError("write your pallas kernel here")


# speedup vs baseline: 7.2808x; 7.2808x over previous
import functools

import jax
import jax.numpy as jnp
from jax import lax
from jax.experimental import pallas as pl
from jax.experimental.pallas import tpu as pltpu
from jax.experimental.pallas import tpu_sc as plsc

N = 10000
E = 320000
D = 128
H = 128
LATENT = 64
NUM_GRAPHS = 64

NC = 2
NS = 16
NW = NC * NS
CH = 128
CPW = 80
EPAD = NW * CPW * CH
NPAD = N + 16


def _sc_agg_body(h_hbm, src_hbm, dst_hbm, zeros_hbm, out_hbm,
                 src_v, dst_v, rows, acc, sem):
    c = lax.axis_index("c")
    s = lax.axis_index("s")
    wid = s * NC + c

    pltpu.sync_copy(src_hbm.at[pl.ds(wid * CPW, CPW)], src_v)
    pltpu.sync_copy(dst_hbm.at[pl.ds(wid * CPW, CPW)], dst_v)

    zpt = 624
    pltpu.sync_copy(zeros_hbm.at[pl.ds(s * zpt, zpt)], acc.at[pl.ds(s * zpt, zpt)])

    @pl.when(s == NS - 1)
    def _():
        pltpu.sync_copy(zeros_hbm.at[pl.ds(NS * zpt, NPAD - NS * zpt)],
                        acc.at[pl.ds(NS * zpt, NPAD - NS * zpt)])

    plsc.subcore_barrier()

    @pl.loop(0, CPW)
    def _chunk(j):
        pltpu.async_copy(h_hbm.at[src_v.at[j]], rows, sem).wait()
        pltpu.sync_copy(rows, acc.at[dst_v.at[j]], add=True)

    plsc.subcore_barrier()
    opt = 624
    pltpu.sync_copy(acc.at[pl.ds(s * opt, opt)],
                    out_hbm.at[c, pl.ds(s * opt, opt)])

    @pl.when(s == 0)
    def _():
        pltpu.sync_copy(acc.at[pl.ds(NS * opt, N - NS * opt)],
                        out_hbm.at[c, pl.ds(NS * opt, N - NS * opt)])


_sc_agg = pl.kernel(
    _sc_agg_body,
    out_type=jax.ShapeDtypeStruct((NC, N, H), jnp.float32),
    mesh=plsc.VectorSubcoreMesh(core_axis_name="c", subcore_axis_name="s",
                                num_cores=NC, num_subcores=NS),
    scratch_types=[
        pltpu.VMEM((CPW, CH), jnp.int32),
        pltpu.VMEM((CPW, CH), jnp.int32),
        pltpu.VMEM((CH, H), jnp.float32),
        pltpu.VMEM_SHARED((NPAD, H), jnp.float32),
        pltpu.SemaphoreType.DMA,
    ],
)


BLK = 400
NBLK = N // BLK


def _mlp_body(h_ref, acc_ref, w1_ref, b1_ref, w2_ref, b2_ref, o_ref):
    z = h_ref[...] + acc_ref[0] + acc_ref[1]
    a = jnp.maximum(
        jnp.dot(z, w1_ref[...], preferred_element_type=jnp.float32) + b1_ref[...],
        0.0)
    o_ref[...] = jnp.maximum(
        jnp.dot(a, w2_ref[...], preferred_element_type=jnp.float32) + b2_ref[...],
        0.0)


_tc_mlp = pl.pallas_call(
    _mlp_body,
    grid=(NBLK,),
    in_specs=[
        pl.BlockSpec((BLK, H), lambda i: (i, 0)),
        pl.BlockSpec((NC, BLK, H), lambda i: (0, i, 0)),
        pl.BlockSpec((H, H), lambda i: (0, 0)),
        pl.BlockSpec((1, H), lambda i: (0, 0)),
        pl.BlockSpec((H, H), lambda i: (0, 0)),
        pl.BlockSpec((1, H), lambda i: (0, 0)),
    ],
    out_specs=pl.BlockSpec((BLK, H), lambda i: (i, 0)),
    out_shape=jax.ShapeDtypeStruct((N, H), jnp.float32),
)


def _mlp_pool_body(h_ref, acc_ref, w1_ref, b1_ref, w2_ref, b2_ref, p_ref,
                   wmu_ref, bmu_ref, wlv_ref, blv_ref,
                   mu_ref, lv_ref, g_acc):
    i = pl.program_id(0)

    @pl.when(i == 0)
    def _():
        g_acc[...] = jnp.zeros_like(g_acc)

    z = h_ref[...] + acc_ref[0] + acc_ref[1]
    a = jnp.maximum(
        jnp.dot(z, w1_ref[...], preferred_element_type=jnp.float32) + b1_ref[...],
        0.0)
    h3 = jnp.maximum(
        jnp.dot(a, w2_ref[...], preferred_element_type=jnp.float32) + b2_ref[...],
        0.0)
    g_acc[...] += lax.dot_general(p_ref[...], h3, (((0,), (0,)), ((), ())),
                                  preferred_element_type=jnp.float32)

    @pl.when(i == NBLK - 1)
    def _():
        g = g_acc[...]
        mu_ref[...] = jnp.dot(g, wmu_ref[...],
                              preferred_element_type=jnp.float32) + bmu_ref[...]
        lv_ref[...] = jnp.dot(g, wlv_ref[...],
                              preferred_element_type=jnp.float32) + blv_ref[...]


_tc_mlp_pool = pl.pallas_call(
    _mlp_pool_body,
    grid=(NBLK,),
    in_specs=[
        pl.BlockSpec((BLK, H), lambda i: (i, 0)),
        pl.BlockSpec((NC, BLK, H), lambda i: (0, i, 0)),
        pl.BlockSpec((H, H), lambda i: (0, 0)),
        pl.BlockSpec((1, H), lambda i: (0, 0)),
        pl.BlockSpec((H, H), lambda i: (0, 0)),
        pl.BlockSpec((1, H), lambda i: (0, 0)),
        pl.BlockSpec((BLK, NUM_GRAPHS), lambda i: (i, 0)),
        pl.BlockSpec((H, LATENT), lambda i: (0, 0)),
        pl.BlockSpec((1, LATENT), lambda i: (0, 0)),
        pl.BlockSpec((H, LATENT), lambda i: (0, 0)),
        pl.BlockSpec((1, LATENT), lambda i: (0, 0)),
    ],
    out_specs=[
        pl.BlockSpec((NUM_GRAPHS, LATENT), lambda i: (0, 0)),
        pl.BlockSpec((NUM_GRAPHS, LATENT), lambda i: (0, 0)),
    ],
    out_shape=[
        jax.ShapeDtypeStruct((NUM_GRAPHS, LATENT), jnp.float32),
        jax.ShapeDtypeStruct((NUM_GRAPHS, LATENT), jnp.float32),
    ],
    scratch_shapes=[pltpu.VMEM((NUM_GRAPHS, H), jnp.float32)],
)


def kernel(x, edge_index, batch, W1_0, b1_0, W2_0, b2_0, W1_1, b1_1, W2_1,
           b2_1, W1_2, b1_2, W2_2, b2_2, W_mu, b_mu, W_lv, b_lv):
    src = edge_index[0]
    dst = edge_index[1]
    npad = EPAD - E
    pad_src = (jnp.arange(npad, dtype=jnp.int32) * 61) % N
    pad_dst = N + (jnp.arange(npad, dtype=jnp.int32) % (NPAD - N))
    src2d = jnp.concatenate([src, pad_src]).reshape(NW * CPW, CH)
    dst2d = jnp.concatenate([dst, pad_dst]).reshape(NW * CPW, CH)
    zeros = jnp.zeros((NPAD, H), jnp.float32)
    p = (batch[:, None] == jnp.arange(NUM_GRAPHS, dtype=jnp.int32)[None, :]
         ).astype(jnp.float32)

    layers = [(W1_0, b1_0.reshape(1, H), W2_0, b2_0.reshape(1, H)),
              (W1_1, b1_1.reshape(1, H), W2_1, b2_1.reshape(1, H)),
              (W1_2, b1_2.reshape(1, H), W2_2, b2_2.reshape(1, H))]

    h = x
    for li in range(2):
        w1, b1, w2, b2 = layers[li]
        acc = _sc_agg(h, src2d, dst2d, zeros)
        h = _tc_mlp(h, acc, w1, b1, w2, b2)

    w1, b1, w2, b2 = layers[2]
    acc = _sc_agg(h, src2d, dst2d, zeros)
    mu, lv = _tc_mlp_pool(h, acc, w1, b1, w2, b2, p,
                          W_mu, b_mu.reshape(1, LATENT),
                          W_lv, b_lv.reshape(1, LATENT))
    return (mu, lv)


# speedup vs baseline: 9.7960x; 1.3455x over previous
import functools

import jax
import jax.numpy as jnp
from jax import lax
from jax.experimental import pallas as pl
from jax.experimental.pallas import tpu as pltpu
from jax.experimental.pallas import tpu_sc as plsc

N = 10000
E = 320000
D = 128
H = 128
LATENT = 64
NUM_GRAPHS = 64

NC = 2
NS = 16
NW = NC * NS
CH = 64
CPW = 160
EPAD = NW * CPW * CH
NPAD = N + 16

NBUF = 4
LAG = 2
IBLK = 8
NIB = 4
UNROLL = IBLK * NIB
BPW = CPW // IBLK


def _sc_agg_body(h_hbm, src_hbm, dst_hbm, zeros_hbm, out_hbm,
                 src_v, dst_v, rows, acc, gsems, ssems, isems):
    c = lax.axis_index("c")
    s = lax.axis_index("s")
    wid = s * NC + c

    zpt = 624
    pltpu.sync_copy(zeros_hbm.at[pl.ds(s * zpt, zpt)], acc.at[pl.ds(s * zpt, zpt)])

    @pl.when(s == NS - 1)
    def _():
        pltpu.sync_copy(zeros_hbm.at[pl.ds(NS * zpt, NPAD - NS * zpt)],
                        acc.at[pl.ds(NS * zpt, NPAD - NS * zpt)])

    plsc.subcore_barrier()

    def _iload(j0, ib):
        base = wid * CPW + j0
        return (pltpu.make_async_copy(src_hbm.at[pl.ds(base, IBLK)],
                                      src_v.at[ib], isems.at[ib]),
                pltpu.make_async_copy(dst_hbm.at[pl.ds(base, IBLK)],
                                      dst_v.at[ib], isems.at[ib]))

    def _gather(j, b, ib, r):
        return pltpu.make_async_copy(h_hbm.at[src_v.at[ib, r]], rows.at[b],
                                     gsems.at[b])

    def _scatter(j, b, ib, r):
        return pltpu.make_async_copy(rows.at[b], acc.at[dst_v.at[ib, r]],
                                     ssems.at[b])

    for k in (0, 1):
        for d in _iload(k * IBLK, k):
            d.start()
    for d in _iload(0, 0):
        d.wait()
    for j in range(NBUF):
        _gather(j, j % NBUF, 0, j).start()

    @pl.loop(0, CPW // UNROLL)
    def _grp(g):
        jbase = g * UNROLL
        for u in range(UNROLL):
            j = jbase + u
            b = u % NBUF
            ib = (u // IBLK) % NIB
            r = u % IBLK

            if r == 0:
                @pl.when(j + 2 * IBLK < CPW)
                def _():
                    for d in _iload(j + 2 * IBLK, (ib + 2) % NIB):
                        d.start()

            _gather(j, b, ib, r).wait()
            _scatter(j, b, ib, r).start(add=True)

            jd = j - LAG
            jg = j - LAG + NBUF
            ud = (u + UNROLL - LAG) % UNROLL
            ug = (u + NBUF - LAG) % UNROLL
            ibg = (ug // IBLK) % NIB
            rg = ug % IBLK

            if rg == 0:
                @pl.when((jg < CPW) & (jg >= IBLK))
                def _():
                    for d in _iload(jg, ibg):
                        d.wait()

            @pl.when(j >= LAG)
            def _():
                _scatter(jd, ud % NBUF, (ud // IBLK) % NIB, ud % IBLK).wait()

                @pl.when(jg < CPW)
                def _():
                    _gather(jg, ug % NBUF, ibg, rg).start()

    for j in range(CPW - LAG, CPW):
        _scatter(j, j % NBUF, (j // IBLK) % NIB, j % IBLK).wait()

    plsc.subcore_barrier()
    opt = 624
    pltpu.sync_copy(acc.at[pl.ds(s * opt, opt)],
                    out_hbm.at[c, pl.ds(s * opt, opt)])

    @pl.when(s == 0)
    def _():
        pltpu.sync_copy(acc.at[pl.ds(NS * opt, N - NS * opt)],
                        out_hbm.at[c, pl.ds(NS * opt, N - NS * opt)])


_sc_agg = pl.kernel(
    _sc_agg_body,
    out_type=jax.ShapeDtypeStruct((NC, N, H), jnp.float32),
    mesh=plsc.VectorSubcoreMesh(core_axis_name="c", subcore_axis_name="s",
                                num_cores=NC, num_subcores=NS),
    scratch_types=[
        pltpu.VMEM((NIB, IBLK, CH), jnp.int32),
        pltpu.VMEM((NIB, IBLK, CH), jnp.int32),
        pltpu.VMEM((NBUF, CH, H), jnp.float32),
        pltpu.VMEM_SHARED((NPAD, H), jnp.float32),
        pltpu.SemaphoreType.DMA((NBUF,)),
        pltpu.SemaphoreType.DMA((NBUF,)),
        pltpu.SemaphoreType.DMA((NIB,)),
    ],
)


BLK = 400
NBLK = N // BLK


def _mlp_body(h_ref, acc_ref, w1_ref, b1_ref, w2_ref, b2_ref, o_ref):
    z = h_ref[...] + acc_ref[0] + acc_ref[1]
    a = jnp.maximum(
        jnp.dot(z, w1_ref[...], preferred_element_type=jnp.float32) + b1_ref[...],
        0.0)
    o_ref[...] = jnp.maximum(
        jnp.dot(a, w2_ref[...], preferred_element_type=jnp.float32) + b2_ref[...],
        0.0)


_tc_mlp = pl.pallas_call(
    _mlp_body,
    grid=(NBLK,),
    in_specs=[
        pl.BlockSpec((BLK, H), lambda i: (i, 0)),
        pl.BlockSpec((NC, BLK, H), lambda i: (0, i, 0)),
        pl.BlockSpec((H, H), lambda i: (0, 0)),
        pl.BlockSpec((1, H), lambda i: (0, 0)),
        pl.BlockSpec((H, H), lambda i: (0, 0)),
        pl.BlockSpec((1, H), lambda i: (0, 0)),
    ],
    out_specs=pl.BlockSpec((BLK, H), lambda i: (i, 0)),
    out_shape=jax.ShapeDtypeStruct((N, H), jnp.float32),
)


def _mlp_pool_body(h_ref, acc_ref, w1_ref, b1_ref, w2_ref, b2_ref, p_ref,
                   wmu_ref, bmu_ref, wlv_ref, blv_ref,
                   mu_ref, lv_ref, g_acc):
    i = pl.program_id(0)

    @pl.when(i == 0)
    def _():
        g_acc[...] = jnp.zeros_like(g_acc)

    z = h_ref[...] + acc_ref[0] + acc_ref[1]
    a = jnp.maximum(
        jnp.dot(z, w1_ref[...], preferred_element_type=jnp.float32) + b1_ref[...],
        0.0)
    h3 = jnp.maximum(
        jnp.dot(a, w2_ref[...], preferred_element_type=jnp.float32) + b2_ref[...],
        0.0)
    g_acc[...] += lax.dot_general(p_ref[...], h3, (((0,), (0,)), ((), ())),
                                  preferred_element_type=jnp.float32)

    @pl.when(i == NBLK - 1)
    def _():
        g = g_acc[...]
        mu_ref[...] = jnp.dot(g, wmu_ref[...],
                              preferred_element_type=jnp.float32) + bmu_ref[...]
        lv_ref[...] = jnp.dot(g, wlv_ref[...],
                              preferred_element_type=jnp.float32) + blv_ref[...]


_tc_mlp_pool = pl.pallas_call(
    _mlp_pool_body,
    grid=(NBLK,),
    in_specs=[
        pl.BlockSpec((BLK, H), lambda i: (i, 0)),
        pl.BlockSpec((NC, BLK, H), lambda i: (0, i, 0)),
        pl.BlockSpec((H, H), lambda i: (0, 0)),
        pl.BlockSpec((1, H), lambda i: (0, 0)),
        pl.BlockSpec((H, H), lambda i: (0, 0)),
        pl.BlockSpec((1, H), lambda i: (0, 0)),
        pl.BlockSpec((BLK, NUM_GRAPHS), lambda i: (i, 0)),
        pl.BlockSpec((H, LATENT), lambda i: (0, 0)),
        pl.BlockSpec((1, LATENT), lambda i: (0, 0)),
        pl.BlockSpec((H, LATENT), lambda i: (0, 0)),
        pl.BlockSpec((1, LATENT), lambda i: (0, 0)),
    ],
    out_specs=[
        pl.BlockSpec((NUM_GRAPHS, LATENT), lambda i: (0, 0)),
        pl.BlockSpec((NUM_GRAPHS, LATENT), lambda i: (0, 0)),
    ],
    out_shape=[
        jax.ShapeDtypeStruct((NUM_GRAPHS, LATENT), jnp.float32),
        jax.ShapeDtypeStruct((NUM_GRAPHS, LATENT), jnp.float32),
    ],
    scratch_shapes=[pltpu.VMEM((NUM_GRAPHS, H), jnp.float32)],
)


def kernel(x, edge_index, batch, W1_0, b1_0, W2_0, b2_0, W1_1, b1_1, W2_1,
           b2_1, W1_2, b1_2, W2_2, b2_2, W_mu, b_mu, W_lv, b_lv):
    src = edge_index[0]
    dst = edge_index[1]
    npad = EPAD - E
    pad_src = (jnp.arange(npad, dtype=jnp.int32) * 61) % N
    pad_dst = N + (jnp.arange(npad, dtype=jnp.int32) % (NPAD - N))
    src2d = jnp.concatenate([src, pad_src]).reshape(NW * CPW, CH)
    dst2d = jnp.concatenate([dst, pad_dst]).reshape(NW * CPW, CH)
    zeros = jnp.zeros((NPAD, H), jnp.float32)
    p = (batch[:, None] == jnp.arange(NUM_GRAPHS, dtype=jnp.int32)[None, :]
         ).astype(jnp.float32)

    layers = [(W1_0, b1_0.reshape(1, H), W2_0, b2_0.reshape(1, H)),
              (W1_1, b1_1.reshape(1, H), W2_1, b2_1.reshape(1, H)),
              (W1_2, b1_2.reshape(1, H), W2_2, b2_2.reshape(1, H))]

    h = x
    for li in range(2):
        w1, b1, w2, b2 = layers[li]
        acc = _sc_agg(h, src2d, dst2d, zeros)
        h = _tc_mlp(h, acc, w1, b1, w2, b2)

    w1, b1, w2, b2 = layers[2]
    acc = _sc_agg(h, src2d, dst2d, zeros)
    mu, lv = _tc_mlp_pool(h, acc, w1, b1, w2, b2, p,
                          W_mu, b_mu.reshape(1, LATENT),
                          W_lv, b_lv.reshape(1, LATENT))
    return (mu, lv)


# speedup vs baseline: 12.9735x; 1.3244x over previous
import jax
import jax.numpy as jnp
from jax import lax
from jax.experimental import pallas as pl
from jax.experimental.pallas import tpu as pltpu
from jax.experimental.pallas import tpu_sc as plsc

N = 10000
E = 320000
D = 128
H = 128
LATENT = 64
NUM_GRAPHS = 64

NC = 2
NS = 16
NW = NC * NS
CH = 64
CPW = 160
EPAD = NW * CPW * CH
NPAD = N + 16

NBUF = 5
LAG = 1
IBLK = 8
NIB = 2
UNROLL = 80


def _sc_agg_body(h_hbm, src_hbm, dst_hbm, zeros_hbm, out_hbm,
                 src_v, dst_v, rows, acc, gsems, ssems, isems):
    c = lax.axis_index("c")
    s = lax.axis_index("s")
    wid = s * NC + c

    def _iload(j0, ib):
        base = wid * CPW + j0
        return (pltpu.make_async_copy(src_hbm.at[pl.ds(base, IBLK)],
                                      src_v.at[ib], isems.at[ib]),
                pltpu.make_async_copy(dst_hbm.at[pl.ds(base, IBLK)],
                                      dst_v.at[ib], isems.at[ib]))

    def _gather(j, b, ib, r):
        return pltpu.make_async_copy(h_hbm.at[src_v.at[ib, r]], rows.at[b],
                                     gsems.at[b])

    def _scatter(j, b, ib, r):
        return pltpu.make_async_copy(rows.at[b], acc.at[dst_v.at[ib, r]],
                                     ssems.at[b])

    for k in (0, 1):
        for d in _iload(k * IBLK, k):
            d.start()
    for d in _iload(0, 0):
        d.wait()
    for j in range(NBUF):
        _gather(j, j % NBUF, 0, j).start()

    zpt = 624
    pltpu.sync_copy(zeros_hbm.at[pl.ds(s * zpt, zpt)], acc.at[pl.ds(s * zpt, zpt)])

    @pl.when(s == NS - 1)
    def _():
        pltpu.sync_copy(zeros_hbm.at[pl.ds(NS * zpt, NPAD - NS * zpt)],
                        acc.at[pl.ds(NS * zpt, NPAD - NS * zpt)])

    plsc.subcore_barrier()

    @pl.loop(0, CPW // UNROLL)
    def _grp(g):
        jbase = g * UNROLL
        for u in range(UNROLL):
            j = jbase + u
            b = u % NBUF
            ib = (u // IBLK) % NIB
            r = u % IBLK

            if r == 2:
                @pl.when((j >= IBLK) & (j + 2 * IBLK - 2 <= CPW))
                def _():
                    for d in _iload(j + IBLK - 2, (ib + 1) % NIB):
                        d.start()

            _gather(j, b, ib, r).wait()
            _scatter(j, b, ib, r).start(add=True)

            jd = j - LAG
            jg = j - LAG + NBUF
            ud = (u + UNROLL - LAG) % UNROLL
            ug = (u + NBUF - LAG) % UNROLL
            ibg = (ug // IBLK) % NIB
            rg = ug % IBLK

            if rg == 0:
                @pl.when((jg < CPW) & (jg >= IBLK))
                def _():
                    for d in _iload(jg, ibg):
                        d.wait()

            @pl.when(j >= LAG)
            def _():
                _scatter(jd, ud % NBUF, (ud // IBLK) % NIB, ud % IBLK).wait()

                @pl.when(jg < CPW)
                def _():
                    _gather(jg, ug % NBUF, ibg, rg).start()

    for j in range(CPW - LAG, CPW):
        _scatter(j, j % NBUF, (j // IBLK) % NIB, j % IBLK).wait()

    plsc.subcore_barrier()
    opt = 624
    pltpu.sync_copy(acc.at[pl.ds(s * opt, opt)],
                    out_hbm.at[c, pl.ds(s * opt, opt)])

    @pl.when(s == 0)
    def _():
        pltpu.sync_copy(acc.at[pl.ds(NS * opt, N - NS * opt)],
                        out_hbm.at[c, pl.ds(NS * opt, N - NS * opt)])


_sc_agg = pl.kernel(
    _sc_agg_body,
    out_type=jax.ShapeDtypeStruct((NC, N, H), jnp.float32),
    mesh=plsc.VectorSubcoreMesh(core_axis_name="c", subcore_axis_name="s",
                                num_cores=NC, num_subcores=NS),
    scratch_types=[
        pltpu.VMEM((NIB, IBLK, CH), jnp.int32),
        pltpu.VMEM((NIB, IBLK, CH), jnp.int32),
        pltpu.VMEM((NBUF, CH, H), jnp.float32),
        pltpu.VMEM_SHARED((NPAD, H), jnp.float32),
        pltpu.SemaphoreType.DMA((NBUF,)),
        pltpu.SemaphoreType.DMA((NBUF,)),
        pltpu.SemaphoreType.DMA((NIB,)),
    ],
)


BLK = 2000
NBLK = N // BLK


def _mlp_body(h_ref, acc_ref, w1_ref, b1_ref, w2_ref, b2_ref, o_ref):
    z = h_ref[...] + acc_ref[0] + acc_ref[1]
    a = jnp.maximum(
        jnp.dot(z, w1_ref[...], preferred_element_type=jnp.float32) + b1_ref[...],
        0.0)
    o_ref[...] = jnp.maximum(
        jnp.dot(a, w2_ref[...], preferred_element_type=jnp.float32) + b2_ref[...],
        0.0)


_tc_mlp = pl.pallas_call(
    _mlp_body,
    grid=(NBLK,),
    in_specs=[
        pl.BlockSpec((BLK, H), lambda i: (i, 0)),
        pl.BlockSpec((NC, BLK, H), lambda i: (0, i, 0)),
        pl.BlockSpec((H, H), lambda i: (0, 0)),
        pl.BlockSpec((1, H), lambda i: (0, 0)),
        pl.BlockSpec((H, H), lambda i: (0, 0)),
        pl.BlockSpec((1, H), lambda i: (0, 0)),
    ],
    out_specs=pl.BlockSpec((BLK, H), lambda i: (i, 0)),
    out_shape=jax.ShapeDtypeStruct((N, H), jnp.float32),
)


def _mlp_pool_body(h_ref, acc_ref, w1_ref, b1_ref, w2_ref, b2_ref, b_ref,
                   wmu_ref, bmu_ref, wlv_ref, blv_ref,
                   mu_ref, lv_ref, g_acc):
    i = pl.program_id(0)

    @pl.when(i == 0)
    def _():
        g_acc[...] = jnp.zeros_like(g_acc)

    z = h_ref[...] + acc_ref[0] + acc_ref[1]
    a = jnp.maximum(
        jnp.dot(z, w1_ref[...], preferred_element_type=jnp.float32) + b1_ref[...],
        0.0)
    h3 = jnp.maximum(
        jnp.dot(a, w2_ref[...], preferred_element_type=jnp.float32) + b2_ref[...],
        0.0)
    pt = (lax.broadcasted_iota(jnp.int32, (NUM_GRAPHS, BLK), 0)
          == jnp.broadcast_to(b_ref[0], (NUM_GRAPHS, BLK))).astype(jnp.float32)
    g_acc[...] += jnp.dot(pt, h3, preferred_element_type=jnp.float32)

    @pl.when(i == NBLK - 1)
    def _():
        g = g_acc[...]
        mu_ref[...] = jnp.dot(g, wmu_ref[...],
                              preferred_element_type=jnp.float32) + bmu_ref[...]
        lv_ref[...] = jnp.dot(g, wlv_ref[...],
                              preferred_element_type=jnp.float32) + blv_ref[...]


_tc_mlp_pool = pl.pallas_call(
    _mlp_pool_body,
    grid=(NBLK,),
    in_specs=[
        pl.BlockSpec((BLK, H), lambda i: (i, 0)),
        pl.BlockSpec((NC, BLK, H), lambda i: (0, i, 0)),
        pl.BlockSpec((H, H), lambda i: (0, 0)),
        pl.BlockSpec((1, H), lambda i: (0, 0)),
        pl.BlockSpec((H, H), lambda i: (0, 0)),
        pl.BlockSpec((1, H), lambda i: (0, 0)),
        pl.BlockSpec((1, 1, BLK), lambda i: (i, 0, 0)),
        pl.BlockSpec((H, LATENT), lambda i: (0, 0)),
        pl.BlockSpec((1, LATENT), lambda i: (0, 0)),
        pl.BlockSpec((H, LATENT), lambda i: (0, 0)),
        pl.BlockSpec((1, LATENT), lambda i: (0, 0)),
    ],
    out_specs=[
        pl.BlockSpec((NUM_GRAPHS, LATENT), lambda i: (0, 0)),
        pl.BlockSpec((NUM_GRAPHS, LATENT), lambda i: (0, 0)),
    ],
    out_shape=[
        jax.ShapeDtypeStruct((NUM_GRAPHS, LATENT), jnp.float32),
        jax.ShapeDtypeStruct((NUM_GRAPHS, LATENT), jnp.float32),
    ],
    scratch_shapes=[pltpu.VMEM((NUM_GRAPHS, H), jnp.float32)],
)


def kernel(x, edge_index, batch, W1_0, b1_0, W2_0, b2_0, W1_1, b1_1, W2_1,
           b2_1, W1_2, b1_2, W2_2, b2_2, W_mu, b_mu, W_lv, b_lv):
    src = edge_index[0]
    dst = edge_index[1]
    npad = EPAD - E
    pad_src = (jnp.arange(npad, dtype=jnp.int32) * 61) % N
    pad_dst = N + (jnp.arange(npad, dtype=jnp.int32) % (NPAD - N))
    src2d = jnp.concatenate([src, pad_src]).reshape(NW * CPW, CH)
    dst2d = jnp.concatenate([dst, pad_dst]).reshape(NW * CPW, CH)
    zeros = jnp.zeros((NPAD, H), jnp.float32)
    batch3d = batch.reshape(NBLK, 1, BLK)

    layers = [(W1_0, b1_0.reshape(1, H), W2_0, b2_0.reshape(1, H)),
              (W1_1, b1_1.reshape(1, H), W2_1, b2_1.reshape(1, H)),
              (W1_2, b1_2.reshape(1, H), W2_2, b2_2.reshape(1, H))]

    h = x
    for li in range(2):
        w1, b1, w2, b2 = layers[li]
        acc = _sc_agg(h, src2d, dst2d, zeros)
        h = _tc_mlp(h, acc, w1, b1, w2, b2)

    w1, b1, w2, b2 = layers[2]
    acc = _sc_agg(h, src2d, dst2d, zeros)
    mu, lv = _tc_mlp_pool(h, acc, w1, b1, w2, b2, batch3d,
                          W_mu, b_mu.reshape(1, LATENT),
                          W_lv, b_lv.reshape(1, LATENT))
    return (mu, lv)
